# Initial kernel scaffold; baseline (speedup 1.0000x reference)
import jax
import jax.numpy as jnp
from jax.experimental import pallas as pl


def kernel(syn0, syn1, pos_u, pos_v, neg_v):
    raise NotImplementedError("write your pallas kernel here")


# speedup vs baseline: 1.4213x; 1.4213x over previous
import functools

import jax
import jax.numpy as jnp
from jax import lax
from jax.experimental import pallas as pl
from jax.experimental.pallas import tpu as pltpu
from jax.experimental.pallas import tpu_sc as plsc

NC = 2
NS = 16
NW = NC * NS
LANES = 16


def _build_sc_scores(V, D, B, K):
    CB = B // NW
    C = 32
    NG = CB // C
    NKROW = (C * K) // 128
    SCORES = C * (K + 1)

    mesh = plsc.VectorSubcoreMesh(core_axis_name="c", subcore_axis_name="s")

    @functools.partial(
        pl.kernel,
        mesh=mesh,
        out_type=jax.ShapeDtypeStruct((B * (K + 1),), jnp.float32),
        scratch_types=[
            pltpu.VMEM((C,), jnp.int32),
            pltpu.VMEM((C,), jnp.int32),
            pltpu.VMEM((C * K,), jnp.int32),
            pltpu.VMEM((C, D), jnp.float32),
            pltpu.VMEM((C, D), jnp.float32),
            pltpu.VMEM((C * K, D), jnp.float32),
            pltpu.VMEM((SCORES,), jnp.float32),
            pltpu.SemaphoreType.DMA,
        ],
        compiler_params=pltpu.CompilerParams(needs_layout_passes=False),
    )
    def sc_scores(syn0_h, syn1_h, pos_u_h, pos_v_h, negv1d_h, out_h,
                  uidx, vidx, nidx, u_rows, v_rows, n_rows, scores, sem):
        cid = lax.axis_index("c")
        sid = lax.axis_index("s")
        wid = sid * NC + cid

        def chunk_body(g, carry):
            b0 = pl.multiple_of(wid * CB + g * C, C)
            i0 = pl.multiple_of((wid * CB + g * C) * K, C * K)
            pltpu.sync_copy(pos_u_h.at[pl.ds(b0, C)], uidx)
            pltpu.sync_copy(pos_v_h.at[pl.ds(b0, C)], vidx)
            pltpu.sync_copy(negv1d_h.at[pl.ds(i0, C * K)], nidx)
            cu = pltpu.async_copy(syn0_h.at[uidx], u_rows, sem)
            cv = pltpu.async_copy(syn1_h.at[vidx], v_rows, sem)
            cns = [
                pltpu.async_copy(
                    syn1_h.at[nidx.at[pl.ds(j * 128, 128)]],
                    n_rows.at[pl.ds(j * 128, 128)],
                    sem,
                )
                for j in range(NKROW)
            ]
            cu.wait()
            cv.wait()
            for c in cns:
                c.wait()

            for h in range(C // LANES):
                rows16 = lax.iota(jnp.int32, LANES) + h * LANES
                nrow = [rows16 * K + k for k in range(K)]

                def d_body(d, accs):
                    d16 = jnp.full((LANES,), d, dtype=jnp.int32)
                    u_el = plsc.load_gather(u_rows, [rows16, d16])
                    t0 = plsc.load_gather(v_rows, [rows16, d16])
                    new = [accs[0] + u_el * t0]
                    for k in range(K):
                        tk = plsc.load_gather(n_rows, [nrow[k], d16])
                        new.append(accs[1 + k] + u_el * tk)
                    return tuple(new)

                accs0 = tuple(
                    jnp.zeros((LANES,), jnp.float32) for _ in range(K + 1)
                )
                accs = lax.fori_loop(0, D, d_body, accs0)
                base = h * (K + 1)
                scores[pl.ds(base * LANES, LANES)] = accs[0]
                for k in range(K):
                    scores[pl.ds((base + 1 + k) * LANES, LANES)] = -accs[1 + k]

            off = pl.multiple_of((wid * NG + g) * SCORES, SCORES)
            pltpu.sync_copy(scores, out_h.at[pl.ds(off, SCORES)])
            return carry

        lax.fori_loop(0, NG, chunk_body, 0)

    return sc_scores


def _loss_body(inv_b, s_ref, o_ref):
    x = s_ref[...]
    ls = jax.nn.log_sigmoid(x)
    o_ref[...] = (-jnp.sum(ls) * inv_b).reshape(1, 1)


def kernel(syn0, syn1, pos_u, pos_v, neg_v):
    V, D = syn0.shape
    B, K = neg_v.shape
    pos_u = pos_u.astype(jnp.int32)
    pos_v = pos_v.astype(jnp.int32)
    negv1d = neg_v.astype(jnp.int32).reshape(B * K)

    sc_scores = _build_sc_scores(V, D, B, K)
    scores = sc_scores(syn0, syn1, pos_u, pos_v, negv1d)

    n = B * (K + 1)
    scores2d = scores.reshape(n // 128, 128)
    loss = pl.pallas_call(
        functools.partial(_loss_body, 1.0 / B),
        out_shape=jax.ShapeDtypeStruct((1, 1), jnp.float32),
    )(scores2d)
    return loss[0, 0]


# speedup vs baseline: 4.6691x; 3.2851x over previous
import functools

import jax
import jax.numpy as jnp
from jax import lax
from jax.experimental import pallas as pl
from jax.experimental.pallas import tpu as pltpu
from jax.experimental.pallas import tpu_sc as plsc

NC = 2
NS = 16
NW = NC * NS
LANES = 16


def _build_sc_scores(V, D, B, K):
    CB = B // NW
    C = 32
    NG = CB // C
    NKROW = (C * K) // 128
    SCORES = C * (K + 1)

    mesh = plsc.VectorSubcoreMesh(core_axis_name="c", subcore_axis_name="s")

    @functools.partial(
        pl.kernel,
        mesh=mesh,
        out_type=jax.ShapeDtypeStruct((B * (K + 1),), jnp.float32),
        scratch_types=[
            pltpu.VMEM((C,), jnp.int32),
            pltpu.VMEM((C,), jnp.int32),
            pltpu.VMEM((C * K,), jnp.int32),
            pltpu.VMEM((C, D), jnp.float32),
            pltpu.VMEM((C, D), jnp.float32),
            pltpu.VMEM((C * K, D), jnp.float32),
            pltpu.VMEM((SCORES, LANES), jnp.float32),
            pltpu.VMEM((SCORES,), jnp.float32),
            pltpu.SemaphoreType.DMA,
        ],
        compiler_params=pltpu.CompilerParams(
            needs_layout_passes=False, use_tc_tiling_on_sc=False
        ),
    )
    def sc_scores(syn0_h, syn1_h, pos_u_h, pos_v_h, negv1d_h, out_h,
                  uidx, vidx, nidx, u_rows, v_rows, n_rows, partial, scores,
                  sem):
        cid = lax.axis_index("c")
        sid = lax.axis_index("s")
        wid = sid * NC + cid

        def chunk_body(g, carry):
            b0 = pl.multiple_of(wid * CB + g * C, C)
            i0 = pl.multiple_of((wid * CB + g * C) * K, C * K)
            pltpu.sync_copy(pos_u_h.at[pl.ds(b0, C)], uidx)
            pltpu.sync_copy(pos_v_h.at[pl.ds(b0, C)], vidx)
            pltpu.sync_copy(negv1d_h.at[pl.ds(i0, C * K)], nidx)
            cu = pltpu.async_copy(syn0_h.at[uidx], u_rows, sem)
            cv = pltpu.async_copy(syn1_h.at[vidx], v_rows, sem)
            cns = [
                pltpu.async_copy(
                    syn1_h.at[nidx.at[pl.ds(j * 128, 128)]],
                    n_rows.at[pl.ds(j * 128, 128)],
                    sem,
                )
                for j in range(NKROW)
            ]
            cu.wait()
            cv.wait()
            for c in cns:
                c.wait()

            NCH = D // LANES
            NT = K + 1

            def b_body(i, carry):
                u = [u_rows[i, pl.ds(c * LANES, LANES)] for c in range(NCH)]
                p0 = i * NT
                acc = u[0] * v_rows[i, pl.ds(0, LANES)]
                for c in range(1, NCH):
                    acc = acc + u[c] * v_rows[i, pl.ds(c * LANES, LANES)]
                partial[p0] = acc
                r = i * K
                for k in range(K):
                    acc = u[0] * n_rows[r + k, pl.ds(0, LANES)]
                    for c in range(1, NCH):
                        acc = acc + u[c] * n_rows[r + k, pl.ds(c * LANES, LANES)]
                    partial[p0 + 1 + k] = -acc
                return carry

            lax.fori_loop(0, C, b_body, 0)

            iota16 = lax.iota(jnp.int32, LANES)
            cols = [jnp.full((LANES,), l, jnp.int32) for l in range(LANES)]

            def r_body(g2, carry):
                rows = g2 * LANES + iota16
                vs = [
                    plsc.load_gather(partial, [rows, cols[l]])
                    for l in range(LANES)
                ]
                while len(vs) > 1:
                    vs = [vs[2 * j] + vs[2 * j + 1]
                          for j in range(len(vs) // 2)]
                scores[pl.ds(g2 * LANES, LANES)] = vs[0]
                return carry

            lax.fori_loop(0, SCORES // LANES, r_body, 0)

            off = pl.multiple_of((wid * NG + g) * SCORES, SCORES)
            pltpu.sync_copy(scores, out_h.at[pl.ds(off, SCORES)])
            return carry

        lax.fori_loop(0, NG, chunk_body, 0)

    return sc_scores


def _loss_body(inv_b, s_ref, o_ref):
    x = s_ref[...]
    ls = jax.nn.log_sigmoid(x)
    o_ref[...] = (-jnp.sum(ls) * inv_b).reshape(1, 1)


def kernel(syn0, syn1, pos_u, pos_v, neg_v):
    V, D = syn0.shape
    B, K = neg_v.shape
    pos_u = pos_u.astype(jnp.int32)
    pos_v = pos_v.astype(jnp.int32)
    negv1d = neg_v.astype(jnp.int32).reshape(B * K)

    sc_scores = _build_sc_scores(V, D, B, K)
    scores = sc_scores(syn0, syn1, pos_u, pos_v, negv1d)

    n = B * (K + 1)
    scores2d = scores.reshape(n // 128, 128)
    loss = pl.pallas_call(
        functools.partial(_loss_body, 1.0 / B),
        out_shape=jax.ShapeDtypeStruct((1, 1), jnp.float32),
    )(scores2d)
    return loss[0, 0]


# speedup vs baseline: 5.4271x; 1.1623x over previous
import functools

import jax
import jax.numpy as jnp
from jax import lax
from jax.experimental import pallas as pl
from jax.experimental.pallas import tpu as pltpu
from jax.experimental.pallas import tpu_sc as plsc

NC = 2
NS = 16
NW = NC * NS
LANES = 16


def _build_sc_scores(V, D, B, K):
    CB = B // NW
    C = 16
    NG = CB // C
    NIDX = C * K
    GSLICE = 80
    NGATHER = NIDX // GSLICE
    SCORES = C * (K + 1)
    NCH = D // LANES
    NT = K + 1

    mesh = plsc.VectorSubcoreMesh(core_axis_name="c", subcore_axis_name="s")

    @functools.partial(
        pl.kernel,
        mesh=mesh,
        out_type=jax.ShapeDtypeStruct((B * NT,), jnp.float32),
        scratch_types=[
            [pltpu.VMEM((C,), jnp.int32) for _ in range(2)],
            [pltpu.VMEM((C,), jnp.int32) for _ in range(2)],
            [pltpu.VMEM((NIDX,), jnp.int32) for _ in range(2)],
            [pltpu.VMEM((C, D), jnp.float32) for _ in range(2)],
            [pltpu.VMEM((C, D), jnp.float32) for _ in range(2)],
            [pltpu.VMEM((NIDX, D), jnp.float32) for _ in range(2)],
            pltpu.VMEM((SCORES, LANES), jnp.float32),
            pltpu.VMEM((SCORES,), jnp.float32),
            [pltpu.SemaphoreType.DMA for _ in range(2)],
        ],
        compiler_params=pltpu.CompilerParams(
            needs_layout_passes=False, use_tc_tiling_on_sc=False
        ),
    )
    def sc_scores(syn0_h, syn1_h, pos_u_h, pos_v_h, negv1d_h, out_h,
                  uidx, vidx, nidx, u_rows, v_rows, n_rows, partial, scores,
                  sem):
        cid = lax.axis_index("c")
        sid = lax.axis_index("s")
        wid = sid * NC + cid

        def issue(g, p):
            b0 = pl.multiple_of(wid * CB + g * C, C)
            i0 = pl.multiple_of((wid * CB + g * C) * K, NIDX)
            pltpu.sync_copy(pos_u_h.at[pl.ds(b0, C)], uidx[p])
            pltpu.sync_copy(pos_v_h.at[pl.ds(b0, C)], vidx[p])
            pltpu.sync_copy(negv1d_h.at[pl.ds(i0, NIDX)], nidx[p])
            pltpu.async_copy(syn0_h.at[uidx[p]], u_rows[p], sem[p])
            pltpu.async_copy(syn1_h.at[vidx[p]], v_rows[p], sem[p])
            for j in range(NGATHER):
                pltpu.async_copy(
                    syn1_h.at[nidx[p].at[pl.ds(j * GSLICE, GSLICE)]],
                    n_rows[p].at[pl.ds(j * GSLICE, GSLICE)],
                    sem[p],
                )

        def wait(p):
            pltpu.make_async_copy(syn0_h.at[uidx[p]], u_rows[p],
                                  sem[p]).wait()
            pltpu.make_async_copy(syn1_h.at[vidx[p]], v_rows[p],
                                  sem[p]).wait()
            for j in range(NGATHER):
                pltpu.make_async_copy(
                    syn1_h.at[nidx[p].at[pl.ds(j * GSLICE, GSLICE)]],
                    n_rows[p].at[pl.ds(j * GSLICE, GSLICE)],
                    sem[p],
                ).wait()

        iota16 = lax.iota(jnp.int32, LANES)
        cols = [jnp.full((LANES,), l, jnp.int32) for l in range(LANES)]

        def compute(g, p):
            ub, vb, nb = u_rows[p], v_rows[p], n_rows[p]

            def b_body(i, carry):
                u = [ub[i, pl.ds(c * LANES, LANES)] for c in range(NCH)]
                p0 = i * NT
                acc = u[0] * vb[i, pl.ds(0, LANES)]
                for c in range(1, NCH):
                    acc = acc + u[c] * vb[i, pl.ds(c * LANES, LANES)]
                partial[p0] = acc
                r = i * K
                for k in range(K):
                    acc = u[0] * nb[r + k, pl.ds(0, LANES)]
                    for c in range(1, NCH):
                        acc = acc + u[c] * nb[r + k, pl.ds(c * LANES, LANES)]
                    partial[p0 + 1 + k] = -acc
                return carry

            lax.fori_loop(0, C, b_body, 0)

            def r_body(g2, carry):
                rows = g2 * LANES + iota16
                vs = [
                    plsc.load_gather(partial, [rows, cols[l]])
                    for l in range(LANES)
                ]
                while len(vs) > 1:
                    vs = [vs[2 * j] + vs[2 * j + 1]
                          for j in range(len(vs) // 2)]
                scores[pl.ds(g2 * LANES, LANES)] = vs[0]
                return carry

            lax.fori_loop(0, SCORES // LANES, r_body, 0)

            off = pl.multiple_of((wid * NG + g) * SCORES, SCORES)
            pltpu.sync_copy(scores, out_h.at[pl.ds(off, SCORES)])

        issue(0, 0)

        def pair_body(j, carry):
            for p in range(2):
                g = j * 2 + p
                wait(p)
                gn = g + 1

                @pl.when(gn < NG)
                def _():
                    issue(gn, 1 - p)

                compute(g, p)
            return carry

        lax.fori_loop(0, NG // 2, pair_body, 0)

    return sc_scores


def _loss_body(inv_b, s_ref, o_ref):
    x = s_ref[...]
    ls = jax.nn.log_sigmoid(x)
    o_ref[...] = (-jnp.sum(ls) * inv_b).reshape(1, 1)


def kernel(syn0, syn1, pos_u, pos_v, neg_v):
    V, D = syn0.shape
    B, K = neg_v.shape
    pos_u = pos_u.astype(jnp.int32)
    pos_v = pos_v.astype(jnp.int32)
    negv1d = neg_v.astype(jnp.int32).reshape(B * K)

    sc_scores = _build_sc_scores(V, D, B, K)
    scores = sc_scores(syn0, syn1, pos_u, pos_v, negv1d)

    n = B * (K + 1)
    scores2d = scores.reshape(n // 128, 128)
    loss = pl.pallas_call(
        functools.partial(_loss_body, 1.0 / B),
        out_shape=jax.ShapeDtypeStruct((1, 1), jnp.float32),
    )(scores2d)
    return loss[0, 0]


# speedup vs baseline: 7.9436x; 1.4637x over previous
import functools

import jax
import jax.numpy as jnp
from jax import lax
from jax.experimental import pallas as pl
from jax.experimental.pallas import tpu as pltpu
from jax.experimental.pallas import tpu_sc as plsc

NC = 2
NS = 16
NW = NC * NS
LANES = 16


def _build_sc_scores(V, D, B, K):
    CB = B // NW
    C = 16
    NG = CB // C
    NIDX = C * K
    GSLICE = 80
    NGATHER = NIDX // GSLICE
    SCORES = C * (K + 1)
    NCH = D // LANES
    NT = K + 1

    mesh = plsc.VectorSubcoreMesh(core_axis_name="c", subcore_axis_name="s")

    @functools.partial(
        pl.kernel,
        mesh=mesh,
        out_type=jax.ShapeDtypeStruct((B * NT,), jnp.float32),
        scratch_types=[
            [pltpu.VMEM((C,), jnp.int32) for _ in range(2)],
            [pltpu.VMEM((C,), jnp.int32) for _ in range(2)],
            [pltpu.VMEM((NIDX,), jnp.int32) for _ in range(2)],
            [pltpu.VMEM((C, D), jnp.float32) for _ in range(2)],
            [pltpu.VMEM((C, D), jnp.float32) for _ in range(2)],
            [pltpu.VMEM((NIDX, D), jnp.float32) for _ in range(2)],
            pltpu.VMEM((SCORES, LANES), jnp.float32),
            pltpu.VMEM((SCORES,), jnp.float32),
            [pltpu.SemaphoreType.DMA for _ in range(2)],
        ],
        compiler_params=pltpu.CompilerParams(
            needs_layout_passes=False, use_tc_tiling_on_sc=False
        ),
    )
    def sc_scores(syn0_h, syn1_h, pos_u_h, pos_v_h, negv1d_h, out_h,
                  uidx, vidx, nidx, u_rows, v_rows, n_rows, partial, scores,
                  sem):
        cid = lax.axis_index("c")
        sid = lax.axis_index("s")
        wid = sid * NC + cid

        def issue(g, p):
            b0 = pl.multiple_of(wid * CB + g * C, C)
            i0 = pl.multiple_of((wid * CB + g * C) * K, NIDX)
            pltpu.sync_copy(pos_u_h.at[pl.ds(b0, C)], uidx[p])
            pltpu.sync_copy(pos_v_h.at[pl.ds(b0, C)], vidx[p])
            pltpu.sync_copy(negv1d_h.at[pl.ds(i0, NIDX)], nidx[p])
            pltpu.async_copy(syn0_h.at[uidx[p]], u_rows[p], sem[p])
            pltpu.async_copy(syn1_h.at[vidx[p]], v_rows[p], sem[p])
            for j in range(NGATHER):
                pltpu.async_copy(
                    syn1_h.at[nidx[p].at[pl.ds(j * GSLICE, GSLICE)]],
                    n_rows[p].at[pl.ds(j * GSLICE, GSLICE)],
                    sem[p],
                )

        def wait(p):
            pltpu.make_async_copy(syn0_h.at[uidx[p]], u_rows[p],
                                  sem[p]).wait()
            pltpu.make_async_copy(syn1_h.at[vidx[p]], v_rows[p],
                                  sem[p]).wait()
            for j in range(NGATHER):
                pltpu.make_async_copy(
                    syn1_h.at[nidx[p].at[pl.ds(j * GSLICE, GSLICE)]],
                    n_rows[p].at[pl.ds(j * GSLICE, GSLICE)],
                    sem[p],
                ).wait()

        iota16 = lax.iota(jnp.int32, LANES)
        cols = [jnp.full((LANES,), l, jnp.int32) for l in range(LANES)]

        def compute(g, p):
            ub, vb, nb = u_rows[p], v_rows[p], n_rows[p]

            def _dot(u, row_ref, r):
                prods = [
                    u[c] * row_ref[r, pl.ds(c * LANES, LANES)]
                    for c in range(NCH)
                ]
                while len(prods) > 1:
                    prods = [prods[2 * j] + prods[2 * j + 1]
                             for j in range(len(prods) // 2)]
                return prods[0]

            @plsc.parallel_loop(0, C, 1, unroll=2)
            def b_body(i):
                u = [ub[i, pl.ds(c * LANES, LANES)] for c in range(NCH)]
                p0 = i * NT
                partial[p0] = _dot(u, vb, i)
                r = i * K
                for k in range(K):
                    partial[p0 + 1 + k] = -_dot(u, nb, r + k)

            @plsc.parallel_loop(0, SCORES // LANES, 1, unroll=2)
            def r_body(g2):
                rows = g2 * LANES + iota16
                vs = [
                    plsc.load_gather(partial, [rows, cols[l]])
                    for l in range(LANES)
                ]
                while len(vs) > 1:
                    vs = [vs[2 * j] + vs[2 * j + 1]
                          for j in range(len(vs) // 2)]
                scores[pl.ds(g2 * LANES, LANES)] = vs[0]

            off = pl.multiple_of((wid * NG + g) * SCORES, SCORES)
            pltpu.sync_copy(scores, out_h.at[pl.ds(off, SCORES)])

        issue(0, 0)

        def pair_body(j, carry):
            for p in range(2):
                g = j * 2 + p
                wait(p)
                gn = g + 1

                @pl.when(gn < NG)
                def _():
                    issue(gn, 1 - p)

                compute(g, p)
            return carry

        lax.fori_loop(0, NG // 2, pair_body, 0)

    return sc_scores


def _loss_body(inv_b, s_ref, o_ref):
    x = s_ref[...]
    ls = jax.nn.log_sigmoid(x)
    o_ref[...] = (-jnp.sum(ls) * inv_b).reshape(1, 1)


def kernel(syn0, syn1, pos_u, pos_v, neg_v):
    V, D = syn0.shape
    B, K = neg_v.shape
    pos_u = pos_u.astype(jnp.int32)
    pos_v = pos_v.astype(jnp.int32)
    negv1d = neg_v.astype(jnp.int32).reshape(B * K)

    sc_scores = _build_sc_scores(V, D, B, K)
    scores = sc_scores(syn0, syn1, pos_u, pos_v, negv1d)

    n = B * (K + 1)
    scores2d = scores.reshape(n // 128, 128)
    loss = pl.pallas_call(
        functools.partial(_loss_body, 1.0 / B),
        out_shape=jax.ShapeDtypeStruct((1, 1), jnp.float32),
    )(scores2d)
    return loss[0, 0]


# speedup vs baseline: 8.6415x; 1.0879x over previous
import functools

import jax
import jax.numpy as jnp
from jax import lax
from jax.experimental import pallas as pl
from jax.experimental.pallas import tpu as pltpu
from jax.experimental.pallas import tpu_sc as plsc

NC = 2
NS = 16
NW = NC * NS
LANES = 16


def _build_sc_scores(V, D, B, K):
    CB = B // NW
    C = 16
    NG = CB // C
    NIDX = C * K
    GSLICE = 80
    NGATHER = NIDX // GSLICE
    SCORES = C * (K + 1)
    NCH = D // LANES
    NT = K + 1

    mesh = plsc.VectorSubcoreMesh(core_axis_name="c", subcore_axis_name="s")

    @functools.partial(
        pl.kernel,
        mesh=mesh,
        out_type=jax.ShapeDtypeStruct((B * NT,), jnp.float32),
        scratch_types=[
            [pltpu.VMEM((C,), jnp.int32) for _ in range(2)],
            [pltpu.VMEM((C,), jnp.int32) for _ in range(2)],
            [pltpu.VMEM((NIDX,), jnp.int32) for _ in range(2)],
            [pltpu.VMEM((C + 1, D), jnp.float32) for _ in range(2)],
            [pltpu.VMEM((C + 1, D), jnp.float32) for _ in range(2)],
            [pltpu.VMEM((NIDX, D), jnp.float32) for _ in range(2)],
            pltpu.VMEM((SCORES, LANES), jnp.float32),
            pltpu.VMEM((SCORES,), jnp.float32),
            [pltpu.SemaphoreType.DMA for _ in range(2)],
        ],
        compiler_params=pltpu.CompilerParams(
            needs_layout_passes=False, use_tc_tiling_on_sc=False
        ),
    )
    def sc_scores(syn0_h, syn1_h, pos_u_h, pos_v_h, negv1d_h, out_h,
                  uidx, vidx, nidx, u_rows, v_rows, n_rows, partial, scores,
                  sem):
        cid = lax.axis_index("c")
        sid = lax.axis_index("s")
        wid = sid * NC + cid

        def issue(g, p):
            b0 = pl.multiple_of(wid * CB + g * C, C)
            i0 = pl.multiple_of((wid * CB + g * C) * K, NIDX)
            pltpu.sync_copy(pos_u_h.at[pl.ds(b0, C)], uidx[p])
            pltpu.sync_copy(pos_v_h.at[pl.ds(b0, C)], vidx[p])
            pltpu.sync_copy(negv1d_h.at[pl.ds(i0, NIDX)], nidx[p])
            pltpu.async_copy(syn0_h.at[uidx[p]],
                             u_rows[p].at[pl.ds(0, C)], sem[p])
            pltpu.async_copy(syn1_h.at[vidx[p]],
                             v_rows[p].at[pl.ds(0, C)], sem[p])
            for j in range(NGATHER):
                pltpu.async_copy(
                    syn1_h.at[nidx[p].at[pl.ds(j * GSLICE, GSLICE)]],
                    n_rows[p].at[pl.ds(j * GSLICE, GSLICE)],
                    sem[p],
                )

        def wait(p):
            pltpu.make_async_copy(syn0_h.at[uidx[p]],
                                  u_rows[p].at[pl.ds(0, C)], sem[p]).wait()
            pltpu.make_async_copy(syn1_h.at[vidx[p]],
                                  v_rows[p].at[pl.ds(0, C)], sem[p]).wait()
            for j in range(NGATHER):
                pltpu.make_async_copy(
                    syn1_h.at[nidx[p].at[pl.ds(j * GSLICE, GSLICE)]],
                    n_rows[p].at[pl.ds(j * GSLICE, GSLICE)],
                    sem[p],
                ).wait()

        iota16 = lax.iota(jnp.int32, LANES)
        cols = [jnp.full((LANES,), l, jnp.int32) for l in range(LANES)]

        def compute(g, p):
            ub, vb, nb = u_rows[p], v_rows[p], n_rows[p]

            def _dot(u, row_ref, r):
                prods = [
                    u[c] * row_ref[r, pl.ds(c * LANES, LANES)]
                    for c in range(NCH)
                ]
                while len(prods) > 1:
                    prods = [prods[2 * j] + prods[2 * j + 1]
                             for j in range(len(prods) // 2)]
                return prods[0]

            @plsc.parallel_loop(0, C, 1, unroll=2)
            def b_body(i):
                u = [ub[i, pl.ds(c * LANES, LANES)] for c in range(NCH)]
                p0 = i * NT
                r = i * K

                def _loads(t):
                    if t == 0:
                        return [vb[i, pl.ds(c * LANES, LANES)]
                                for c in range(NCH)]
                    return [nb[r + t - 1, pl.ds(c * LANES, LANES)]
                            for c in range(NCH)]

                def _tree(t, rows):
                    prods = [u[c] * rows[c] for c in range(NCH)]
                    while len(prods) > 1:
                        prods = [prods[2 * j] + prods[2 * j + 1]
                                 for j in range(len(prods) // 2)]
                    partial[p0 + t] = prods[0] if t == 0 else -prods[0]

                pending = _loads(0)
                for t in range(NT):
                    nxt = _loads(t + 1) if t + 1 < NT else None
                    _tree(t, pending)
                    pending = nxt

            @plsc.parallel_loop(0, SCORES // LANES, 1, unroll=2)
            def r_body(g2):
                rows = g2 * LANES + iota16
                vs = [
                    plsc.load_gather(partial, [rows, cols[l]])
                    for l in range(LANES)
                ]
                while len(vs) > 1:
                    vs = [vs[2 * j] + vs[2 * j + 1]
                          for j in range(len(vs) // 2)]
                scores[pl.ds(g2 * LANES, LANES)] = vs[0]

            off = pl.multiple_of((wid * NG + g) * SCORES, SCORES)
            pltpu.sync_copy(scores, out_h.at[pl.ds(off, SCORES)])

        issue(0, 0)

        def pair_body(j, carry):
            for p in range(2):
                g = j * 2 + p
                wait(p)
                gn = g + 1

                @pl.when(gn < NG)
                def _():
                    issue(gn, 1 - p)

                compute(g, p)
            return carry

        lax.fori_loop(0, NG // 2, pair_body, 0)

    return sc_scores


def _loss_body(inv_b, s_ref, o_ref):
    x = s_ref[...]
    ls = jax.nn.log_sigmoid(x)
    o_ref[...] = (-jnp.sum(ls) * inv_b).reshape(1, 1)


def kernel(syn0, syn1, pos_u, pos_v, neg_v):
    V, D = syn0.shape
    B, K = neg_v.shape
    pos_u = pos_u.astype(jnp.int32)
    pos_v = pos_v.astype(jnp.int32)
    negv1d = neg_v.astype(jnp.int32).reshape(B * K)

    sc_scores = _build_sc_scores(V, D, B, K)
    scores = sc_scores(syn0, syn1, pos_u, pos_v, negv1d)

    n = B * (K + 1)
    scores2d = scores.reshape(n // 128, 128)
    loss = pl.pallas_call(
        functools.partial(_loss_body, 1.0 / B),
        out_shape=jax.ShapeDtypeStruct((1, 1), jnp.float32),
    )(scores2d)
    return loss[0, 0]


# speedup vs baseline: 8.7641x; 1.0142x over previous
import functools

import jax
import jax.numpy as jnp
from jax import lax
from jax.experimental import pallas as pl
from jax.experimental.pallas import tpu as pltpu
from jax.experimental.pallas import tpu_sc as plsc

NC = 2
NS = 16
NW = NC * NS
LANES = 16


def _build_sc_scores(V, D, B, K):
    CB = B // NW
    C = 16
    NG = CB // C
    NIDX = C * K
    GSLICE = 80
    NGATHER = NIDX // GSLICE
    SCORES = C * (K + 1)
    NCH = D // LANES
    NT = K + 1

    mesh = plsc.VectorSubcoreMesh(core_axis_name="c", subcore_axis_name="s")

    @functools.partial(
        pl.kernel,
        mesh=mesh,
        out_type=jax.ShapeDtypeStruct((B * NT,), jnp.float32),
        scratch_types=[
            [pltpu.VMEM((C,), jnp.int32) for _ in range(2)],
            [pltpu.VMEM((C,), jnp.int32) for _ in range(2)],
            [pltpu.VMEM((NIDX,), jnp.int32) for _ in range(2)],
            [pltpu.VMEM((C + 1, D), jnp.float32) for _ in range(2)],
            [pltpu.VMEM((C + 1, D), jnp.float32) for _ in range(2)],
            [pltpu.VMEM((NIDX, D), jnp.float32) for _ in range(2)],
            pltpu.VMEM((SCORES, LANES), jnp.float32),
            pltpu.VMEM((SCORES,), jnp.float32),
            [pltpu.SemaphoreType.DMA for _ in range(2)],
        ],
        compiler_params=pltpu.CompilerParams(
            needs_layout_passes=False, use_tc_tiling_on_sc=False
        ),
    )
    def sc_scores(syn0_h, syn1_h, pos_u_h, pos_v_h, negv1d_h, out_h,
                  uidx, vidx, nidx, u_rows, v_rows, n_rows, partial, scores,
                  sem):
        cid = lax.axis_index("c")
        sid = lax.axis_index("s")
        wid = sid * NC + cid

        def issue(g, p):
            b0 = pl.multiple_of(wid * CB + g * C, C)
            i0 = pl.multiple_of((wid * CB + g * C) * K, NIDX)
            pltpu.sync_copy(pos_u_h.at[pl.ds(b0, C)], uidx[p])
            pltpu.sync_copy(pos_v_h.at[pl.ds(b0, C)], vidx[p])
            pltpu.sync_copy(negv1d_h.at[pl.ds(i0, NIDX)], nidx[p])
            pltpu.async_copy(syn0_h.at[uidx[p]],
                             u_rows[p].at[pl.ds(0, C)], sem[p])
            pltpu.async_copy(syn1_h.at[vidx[p]],
                             v_rows[p].at[pl.ds(0, C)], sem[p])
            for j in range(NGATHER):
                pltpu.async_copy(
                    syn1_h.at[nidx[p].at[pl.ds(j * GSLICE, GSLICE)]],
                    n_rows[p].at[pl.ds(j * GSLICE, GSLICE)],
                    sem[p],
                )

        def wait(p):
            pltpu.make_async_copy(syn0_h.at[uidx[p]],
                                  u_rows[p].at[pl.ds(0, C)], sem[p]).wait()
            pltpu.make_async_copy(syn1_h.at[vidx[p]],
                                  v_rows[p].at[pl.ds(0, C)], sem[p]).wait()
            for j in range(NGATHER):
                pltpu.make_async_copy(
                    syn1_h.at[nidx[p].at[pl.ds(j * GSLICE, GSLICE)]],
                    n_rows[p].at[pl.ds(j * GSLICE, GSLICE)],
                    sem[p],
                ).wait()

        iota16 = lax.iota(jnp.int32, LANES)
        cols = [jnp.full((LANES,), l, jnp.int32) for l in range(LANES)]

        def compute(g, p):
            ub, vb, nb = u_rows[p], v_rows[p], n_rows[p]

            def _dot(u, row_ref, r):
                prods = [
                    u[c] * row_ref[r, pl.ds(c * LANES, LANES)]
                    for c in range(NCH)
                ]
                while len(prods) > 1:
                    prods = [prods[2 * j] + prods[2 * j + 1]
                             for j in range(len(prods) // 2)]
                return prods[0]

            @plsc.parallel_loop(0, C, 1, unroll=2)
            def b_body(i):
                u = [ub[i, pl.ds(c * LANES, LANES)] for c in range(NCH)]
                p0 = i * NT
                r = i * K

                def _loads(t):
                    if t == 0:
                        return [vb[i, pl.ds(c * LANES, LANES)]
                                for c in range(NCH)]
                    return [nb[r + t - 1, pl.ds(c * LANES, LANES)]
                            for c in range(NCH)]

                def _tree(t, rows):
                    prods = [u[c] * rows[c] for c in range(NCH)]
                    while len(prods) > 1:
                        prods = [prods[2 * j] + prods[2 * j + 1]
                                 for j in range(len(prods) // 2)]
                    partial[p0 + t] = prods[0] if t == 0 else -prods[0]

                pend0 = _loads(0)
                pend1 = _loads(1)
                for t in range(NT):
                    nxt = _loads(t + 2) if t + 2 < NT else None
                    _tree(t, pend0)
                    pend0, pend1 = pend1, nxt

            @plsc.parallel_loop(0, SCORES // LANES, 1, unroll=2)
            def r_body(g2):
                rows = g2 * LANES + iota16
                vs = [
                    plsc.load_gather(partial, [rows, cols[l]])
                    for l in range(LANES)
                ]
                while len(vs) > 1:
                    vs = [vs[2 * j] + vs[2 * j + 1]
                          for j in range(len(vs) // 2)]
                scores[pl.ds(g2 * LANES, LANES)] = vs[0]

            off = pl.multiple_of((wid * NG + g) * SCORES, SCORES)
            pltpu.sync_copy(scores, out_h.at[pl.ds(off, SCORES)])

        issue(0, 0)

        def pair_body(j, carry):
            for p in range(2):
                g = j * 2 + p
                wait(p)
                gn = g + 1

                @pl.when(gn < NG)
                def _():
                    issue(gn, 1 - p)

                compute(g, p)
            return carry

        lax.fori_loop(0, NG // 2, pair_body, 0)

    return sc_scores


def _loss_body(inv_b, s_ref, o_ref):
    x = s_ref[...]
    ls = jax.nn.log_sigmoid(x)
    o_ref[...] = (-jnp.sum(ls) * inv_b).reshape(1, 1)


def kernel(syn0, syn1, pos_u, pos_v, neg_v):
    V, D = syn0.shape
    B, K = neg_v.shape
    pos_u = pos_u.astype(jnp.int32)
    pos_v = pos_v.astype(jnp.int32)
    negv1d = neg_v.astype(jnp.int32).reshape(B * K)

    sc_scores = _build_sc_scores(V, D, B, K)
    scores = sc_scores(syn0, syn1, pos_u, pos_v, negv1d)

    n = B * (K + 1)
    scores2d = scores.reshape(n // 128, 128)
    loss = pl.pallas_call(
        functools.partial(_loss_body, 1.0 / B),
        out_shape=jax.ShapeDtypeStruct((1, 1), jnp.float32),
    )(scores2d)
    return loss[0, 0]


# speedup vs baseline: 11.5676x; 1.3199x over previous
import functools

import jax
import jax.numpy as jnp
from jax import lax
from jax.experimental import pallas as pl
from jax.experimental.pallas import tpu as pltpu
from jax.experimental.pallas import tpu_sc as plsc

NC = 2
NS = 16
NW = NC * NS
LANES = 16


def _build_sc_scores(V, D, B, K):
    CB = B // NW
    C = 16
    NG = CB // C
    NIDX = C * K
    GSLICE = 80
    NGATHER = NIDX // GSLICE
    SCORES = C * (K + 1)
    NCH = D // LANES
    NT = K + 1

    mesh = plsc.VectorSubcoreMesh(core_axis_name="c", subcore_axis_name="s")

    @functools.partial(
        pl.kernel,
        mesh=mesh,
        out_type=jax.ShapeDtypeStruct((B * NT,), jnp.float32),
        scratch_types=[
            pltpu.VMEM((CB,), jnp.int32),
            pltpu.VMEM((CB,), jnp.int32),
            pltpu.VMEM((CB * K,), jnp.int32),
            [pltpu.VMEM((C + 1, D), jnp.float32) for _ in range(2)],
            [pltpu.VMEM((C + 1, D), jnp.float32) for _ in range(2)],
            [pltpu.VMEM((NIDX, D), jnp.float32) for _ in range(2)],
            pltpu.VMEM((SCORES, LANES), jnp.float32),
            [pltpu.VMEM((SCORES,), jnp.float32) for _ in range(2)],
            [pltpu.SemaphoreType.DMA for _ in range(2)],
            [pltpu.SemaphoreType.DMA for _ in range(2)],
        ],
        compiler_params=pltpu.CompilerParams(
            needs_layout_passes=False, use_tc_tiling_on_sc=False
        ),
    )
    def sc_scores(syn0_h, syn1_h, pos_u_h, pos_v_h, negv1d_h, out_h,
                  uidx, vidx, nidx, u_rows, v_rows, n_rows, partial, scores,
                  sem, semo):
        cid = lax.axis_index("c")
        sid = lax.axis_index("s")
        wid = sid * NC + cid

        b0 = pl.multiple_of(wid * CB, CB)
        pltpu.sync_copy(pos_u_h.at[pl.ds(b0, CB)], uidx)
        pltpu.sync_copy(pos_v_h.at[pl.ds(b0, CB)], vidx)
        pltpu.sync_copy(negv1d_h.at[pl.ds(b0 * K, CB * K)], nidx)

        def issue(g, p):
            l0 = pl.multiple_of(g * C, C)
            n0 = pl.multiple_of(g * NIDX, NIDX)
            pltpu.async_copy(syn0_h.at[uidx.at[pl.ds(l0, C)]],
                             u_rows[p].at[pl.ds(0, C)], sem[p])
            pltpu.async_copy(syn1_h.at[vidx.at[pl.ds(l0, C)]],
                             v_rows[p].at[pl.ds(0, C)], sem[p])
            for j in range(NGATHER):
                pltpu.async_copy(
                    syn1_h.at[nidx.at[pl.ds(n0 + j * GSLICE, GSLICE)]],
                    n_rows[p].at[pl.ds(j * GSLICE, GSLICE)],
                    sem[p],
                )

        def wait(p):
            pltpu.make_async_copy(syn0_h.at[uidx.at[pl.ds(0, C)]],
                                  u_rows[p].at[pl.ds(0, C)], sem[p]).wait()
            pltpu.make_async_copy(syn1_h.at[vidx.at[pl.ds(0, C)]],
                                  v_rows[p].at[pl.ds(0, C)], sem[p]).wait()
            for j in range(NGATHER):
                pltpu.make_async_copy(
                    syn1_h.at[nidx.at[pl.ds(j * GSLICE, GSLICE)]],
                    n_rows[p].at[pl.ds(j * GSLICE, GSLICE)],
                    sem[p],
                ).wait()

        iota16 = lax.iota(jnp.int32, LANES)
        cols = [jnp.full((LANES,), l, jnp.int32) for l in range(LANES)]

        def compute(g, p):
            ub, vb, nb = u_rows[p], v_rows[p], n_rows[p]

            def _dot(u, row_ref, r):
                prods = [
                    u[c] * row_ref[r, pl.ds(c * LANES, LANES)]
                    for c in range(NCH)
                ]
                while len(prods) > 1:
                    prods = [prods[2 * j] + prods[2 * j + 1]
                             for j in range(len(prods) // 2)]
                return prods[0]

            @plsc.parallel_loop(0, C, 1, unroll=2)
            def b_body(i):
                u = [ub[i, pl.ds(c * LANES, LANES)] for c in range(NCH)]
                p0 = i * NT
                r = i * K

                def _loads(t):
                    if t == 0:
                        return [vb[i, pl.ds(c * LANES, LANES)]
                                for c in range(NCH)]
                    return [nb[r + t - 1, pl.ds(c * LANES, LANES)]
                            for c in range(NCH)]

                def _tree(t, rows):
                    prods = [u[c] * rows[c] for c in range(NCH)]
                    while len(prods) > 1:
                        prods = [prods[2 * j] + prods[2 * j + 1]
                                 for j in range(len(prods) // 2)]
                    partial[p0 + t] = prods[0] if t == 0 else -prods[0]

                pend0 = _loads(0)
                pend1 = _loads(1)
                for t in range(NT):
                    nxt = _loads(t + 2) if t + 2 < NT else None
                    _tree(t, pend0)
                    pend0, pend1 = pend1, nxt

            sbuf = scores[p]

            @plsc.parallel_loop(0, SCORES // LANES, 1, unroll=2)
            def r_body(g2):
                rows = g2 * LANES + iota16
                vs = [
                    plsc.load_gather(partial, [rows, cols[l]])
                    for l in range(LANES)
                ]
                while len(vs) > 1:
                    vs = [vs[2 * j] + vs[2 * j + 1]
                          for j in range(len(vs) // 2)]
                sbuf[pl.ds(g2 * LANES, LANES)] = vs[0]

            off = pl.multiple_of((wid * NG + g) * SCORES, SCORES)
            pltpu.async_copy(sbuf, out_h.at[pl.ds(off, SCORES)], semo[p])

        issue(0, 0)

        def pair_body(j, carry):
            for p in range(2):
                g = j * 2 + p
                wait(p)
                gn = g + 1

                @pl.when(gn < NG)
                def _():
                    issue(gn, 1 - p)

                @pl.when(g >= 2)
                def _():
                    off0 = pl.multiple_of(wid * NG * SCORES, SCORES)
                    pltpu.make_async_copy(
                        scores[p], out_h.at[pl.ds(off0, SCORES)], semo[p]
                    ).wait()

                compute(g, p)
            return carry

        lax.fori_loop(0, NG // 2, pair_body, 0)

        for p in range(2):
            off0 = pl.multiple_of(wid * NG * SCORES, SCORES)
            pltpu.make_async_copy(
                scores[p], out_h.at[pl.ds(off0, SCORES)], semo[p]
            ).wait()

    return sc_scores


def _loss_body(inv_b, s_ref, o_ref):
    x = s_ref[...]
    ls = jax.nn.log_sigmoid(x)
    o_ref[...] = (-jnp.sum(ls) * inv_b).reshape(1, 1)


def kernel(syn0, syn1, pos_u, pos_v, neg_v):
    V, D = syn0.shape
    B, K = neg_v.shape
    pos_u = pos_u.astype(jnp.int32)
    pos_v = pos_v.astype(jnp.int32)
    negv1d = neg_v.astype(jnp.int32).reshape(B * K)

    sc_scores = _build_sc_scores(V, D, B, K)
    scores = sc_scores(syn0, syn1, pos_u, pos_v, negv1d)

    n = B * (K + 1)
    scores2d = scores.reshape(n // 128, 128)
    loss = pl.pallas_call(
        functools.partial(_loss_body, 1.0 / B),
        out_shape=jax.ShapeDtypeStruct((1, 1), jnp.float32),
    )(scores2d)
    return loss[0, 0]


# speedup vs baseline: 11.5795x; 1.0010x over previous
import functools

import jax
import jax.numpy as jnp
from jax import lax
from jax.experimental import pallas as pl
from jax.experimental.pallas import tpu as pltpu
from jax.experimental.pallas import tpu_sc as plsc

NC = 2
NS = 16
NW = NC * NS
LANES = 16


def _build_sc_scores(V, D, B, K):
    CB = B // NW
    C = 16
    NG = CB // C
    NIDX = C * K
    GSLICE = 80
    NGATHER = NIDX // GSLICE
    SCORES = C * (K + 1)
    NCH = D // LANES
    NT = K + 1

    mesh = plsc.VectorSubcoreMesh(core_axis_name="c", subcore_axis_name="s")

    @functools.partial(
        pl.kernel,
        mesh=mesh,
        out_type=jax.ShapeDtypeStruct((NW * LANES,), jnp.float32),
        scratch_types=[
            pltpu.VMEM((CB,), jnp.int32),
            pltpu.VMEM((CB,), jnp.int32),
            pltpu.VMEM((CB * K,), jnp.int32),
            [pltpu.VMEM((C + 1, D), jnp.float32) for _ in range(2)],
            [pltpu.VMEM((C + 1, D), jnp.float32) for _ in range(2)],
            [pltpu.VMEM((NIDX, D), jnp.float32) for _ in range(2)],
            pltpu.VMEM((SCORES, LANES), jnp.float32),
            pltpu.VMEM((LANES,), jnp.float32),
            [pltpu.SemaphoreType.DMA for _ in range(2)],
        ],
        compiler_params=pltpu.CompilerParams(
            needs_layout_passes=False, use_tc_tiling_on_sc=False
        ),
    )
    def sc_scores(syn0_h, syn1_h, pos_u_h, pos_v_h, negv1d_h, out_h,
                  uidx, vidx, nidx, u_rows, v_rows, n_rows, partial, tacc,
                  sem):
        cid = lax.axis_index("c")
        sid = lax.axis_index("s")
        wid = sid * NC + cid

        b0 = pl.multiple_of(wid * CB, CB)
        pltpu.sync_copy(pos_u_h.at[pl.ds(b0, CB)], uidx)
        pltpu.sync_copy(pos_v_h.at[pl.ds(b0, CB)], vidx)
        pltpu.sync_copy(negv1d_h.at[pl.ds(b0 * K, CB * K)], nidx)

        def issue(g, p):
            l0 = pl.multiple_of(g * C, C)
            n0 = pl.multiple_of(g * NIDX, NIDX)
            pltpu.async_copy(syn0_h.at[uidx.at[pl.ds(l0, C)]],
                             u_rows[p].at[pl.ds(0, C)], sem[p])
            pltpu.async_copy(syn1_h.at[vidx.at[pl.ds(l0, C)]],
                             v_rows[p].at[pl.ds(0, C)], sem[p])
            for j in range(NGATHER):
                pltpu.async_copy(
                    syn1_h.at[nidx.at[pl.ds(n0 + j * GSLICE, GSLICE)]],
                    n_rows[p].at[pl.ds(j * GSLICE, GSLICE)],
                    sem[p],
                )

        def wait(p):
            pltpu.make_async_copy(syn0_h.at[uidx.at[pl.ds(0, C)]],
                                  u_rows[p].at[pl.ds(0, C)], sem[p]).wait()
            pltpu.make_async_copy(syn1_h.at[vidx.at[pl.ds(0, C)]],
                                  v_rows[p].at[pl.ds(0, C)], sem[p]).wait()
            for j in range(NGATHER):
                pltpu.make_async_copy(
                    syn1_h.at[nidx.at[pl.ds(j * GSLICE, GSLICE)]],
                    n_rows[p].at[pl.ds(j * GSLICE, GSLICE)],
                    sem[p],
                ).wait()

        iota16 = lax.iota(jnp.int32, LANES)
        cols = [jnp.full((LANES,), l, jnp.int32) for l in range(LANES)]
        tacc[...] = jnp.zeros((LANES,), jnp.float32)

        LN2_2P23 = 0.6931471805599453 / 8388608.0

        def _log_sigmoid16(x):
            z = jnp.exp(-jnp.abs(x))
            w = 1.0 + z
            bits = lax.bitcast_convert_type(w, jnp.int32) - 0x3F800000
            y = bits.astype(jnp.float32) * LN2_2P23
            for _ in range(3):
                y = y - 1.0 + w * jnp.exp(-y)
            return jnp.minimum(x, 0.0) - y

        def compute(g, p):
            ub, vb, nb = u_rows[p], v_rows[p], n_rows[p]

            def _dot(u, row_ref, r):
                prods = [
                    u[c] * row_ref[r, pl.ds(c * LANES, LANES)]
                    for c in range(NCH)
                ]
                while len(prods) > 1:
                    prods = [prods[2 * j] + prods[2 * j + 1]
                             for j in range(len(prods) // 2)]
                return prods[0]

            @plsc.parallel_loop(0, C, 1, unroll=2)
            def b_body(i):
                u = [ub[i, pl.ds(c * LANES, LANES)] for c in range(NCH)]
                p0 = i * NT
                r = i * K

                def _loads(t):
                    if t == 0:
                        return [vb[i, pl.ds(c * LANES, LANES)]
                                for c in range(NCH)]
                    return [nb[r + t - 1, pl.ds(c * LANES, LANES)]
                            for c in range(NCH)]

                def _tree(t, rows):
                    prods = [u[c] * rows[c] for c in range(NCH)]
                    while len(prods) > 1:
                        prods = [prods[2 * j] + prods[2 * j + 1]
                                 for j in range(len(prods) // 2)]
                    partial[p0 + t] = prods[0] if t == 0 else -prods[0]

                pend0 = _loads(0)
                pend1 = _loads(1)
                for t in range(NT):
                    nxt = _loads(t + 2) if t + 2 < NT else None
                    _tree(t, pend0)
                    pend0, pend1 = pend1, nxt

            zero16 = jnp.zeros((LANES,), jnp.float32)

            @plsc.parallel_loop(0, SCORES // LANES, 1, unroll=2,
                                carry=zero16)
            def r_body(g2, acc):
                rows = g2 * LANES + iota16
                vs = [
                    plsc.load_gather(partial, [rows, cols[l]])
                    for l in range(LANES)
                ]
                while len(vs) > 1:
                    vs = [vs[2 * j] + vs[2 * j + 1]
                          for j in range(len(vs) // 2)]
                return acc + _log_sigmoid16(vs[0])

            tacc[...] = tacc[...] + r_body

        issue(0, 0)

        def pair_body(j, carry):
            for p in range(2):
                g = j * 2 + p
                wait(p)
                gn = g + 1

                @pl.when(gn < NG)
                def _():
                    issue(gn, 1 - p)

                compute(g, p)
            return carry

        lax.fori_loop(0, NG // 2, pair_body, 0)

        pltpu.sync_copy(tacc, out_h.at[pl.ds(wid * LANES, LANES)])

    return sc_scores


def kernel(syn0, syn1, pos_u, pos_v, neg_v):
    V, D = syn0.shape
    B, K = neg_v.shape
    pos_u = pos_u.astype(jnp.int32)
    pos_v = pos_v.astype(jnp.int32)
    negv1d = neg_v.astype(jnp.int32).reshape(B * K)

    sc_scores = _build_sc_scores(V, D, B, K)
    psums = sc_scores(syn0, syn1, pos_u, pos_v, negv1d)
    return -jnp.sum(psums) / B
